# Initial kernel scaffold; baseline (speedup 1.0000x reference)
import jax
import jax.numpy as jnp
from jax.experimental import pallas as pl


def kernel(x, edge_index, W1, b1, w_mu, w_log_sigma, eps_w, b_mu, b_log_sigma, eps_b):
    raise NotImplementedError("write your pallas kernel here")


# speedup vs baseline: 20.3665x; 20.3665x over previous
import functools

import jax
import jax.numpy as jnp
from jax import lax
from jax.experimental import pallas as pl
from jax.experimental.pallas import tpu as pltpu
from jax.experimental.pallas import tpu_sc as plsc

NC = 2
NS = 16
NW = NC * NS
CHUNK = 128
NPAD = 10240
RPT = NPAD // NS

@functools.lru_cache(maxsize=None)
def _mesh():
    return plsc.VectorSubcoreMesh(core_axis_name="c", subcore_axis_name="s",
                                  num_cores=NC, num_subcores=NS)


def _deg_body(cpt, dst_hbm, out_hbm, deg_sh, idx_v, ones_v, zb_v, tmp_v):
    cid = lax.axis_index("c")
    sid = lax.axis_index("s")
    wid = cid * NS + sid
    z16 = jnp.zeros((16,), jnp.float32)
    for k in range(8):
        zb_v[pl.ds(k * 16, 16)] = z16
        ones_v[pl.ds(k * 16, 16)] = z16 + 1.0
    base = sid * RPT
    for off in range(0, RPT, CHUNK):
        pltpu.sync_copy(zb_v, deg_sh.at[pl.ds(base + off, CHUNK)])
    plsc.subcore_barrier()
    pltpu.sync_copy(dst_hbm.at[wid], idx_v)

    def body(j, carry):
        pltpu.sync_copy(ones_v, deg_sh.at[idx_v.at[j]], add=True)
        return carry

    lax.fori_loop(0, cpt, body, 0)
    plsc.subcore_barrier()
    pltpu.sync_copy(deg_sh.at[pl.ds(base, RPT)], tmp_v)
    pltpu.sync_copy(tmp_v, out_hbm.at[cid].at[pl.ds(base, RPT)])


def _scat_body(cpt, src_hbm, dst_hbm, g_hbm, out_hbm,
               acc_sh, isrc_v, idst_v, rows_v, zb_v, sem):
    cid = lax.axis_index("c")
    sid = lax.axis_index("s")
    wid = cid * NS + sid
    z16 = jnp.zeros((16,), jnp.float32)
    for r in range(40):
        for cc in range(8):
            zb_v[r, pl.ds(cc * 16, 16)] = z16
    base = sid * RPT

    def zcopy(i, carry):
        pltpu.sync_copy(zb_v, acc_sh.at[pl.ds(base + i * 40, 40)])
        return carry

    lax.fori_loop(0, RPT // 40, zcopy, 0)
    plsc.subcore_barrier()
    pltpu.sync_copy(src_hbm.at[wid], isrc_v)
    pltpu.sync_copy(dst_hbm.at[wid], idst_v)

    def body(j, carry):
        pltpu.async_copy(g_hbm.at[isrc_v.at[j]], rows_v, sem).wait()
        pltpu.sync_copy(rows_v, acc_sh.at[idst_v.at[j]], add=True)
        return carry

    lax.fori_loop(0, cpt, body, 0)
    plsc.subcore_barrier()

    def outcp(i, carry):
        pltpu.sync_copy(acc_sh.at[pl.ds(base + i * CHUNK, CHUNK)], rows_v)
        pltpu.sync_copy(rows_v, out_hbm.at[cid].at[pl.ds(base + i * CHUNK, CHUNK)])
        return carry

    lax.fori_loop(0, RPT // CHUNK, outcp, 0)


def _mm_body(x_ref, w_ref, degp_ref, g_ref):
    deg = degp_ref[0, :] + degp_ref[1, :] + 1.0
    dis = lax.rsqrt(deg)
    h = jnp.dot(x_ref[...], w_ref[...], preferred_element_type=jnp.float32)
    g_ref[...] = h * dis[:, None]


def _final_body(s_ref, degp_ref, g_ref, b1_ref, wmu_ref, wls_ref, epsw_ref,
                bmu_ref, bls_ref, epsb_ref, o_ref):
    deg = degp_ref[0, :] + degp_ref[1, :] + 1.0
    dis = lax.rsqrt(deg)
    pre = dis[:, None] * (s_ref[0] + s_ref[1] + g_ref[...]) + b1_ref[...]
    r = jnp.maximum(pre, 0.0)
    w = wmu_ref[...] + jnp.exp(wls_ref[...]) * epsw_ref[...]
    b = bmu_ref[...] + jnp.exp(bls_ref[...]) * epsb_ref[...]
    logits = jax.lax.dot_general(
        r, w, (((1,), (1,)), ((), ())),
        preferred_element_type=jnp.float32) + b
    m = jnp.max(logits, axis=1, keepdims=True)
    e = jnp.exp(logits - m)
    lse = m + jnp.log(jnp.sum(e, axis=1, keepdims=True))
    o_ref[...] = logits - lse


def kernel(x, edge_index, W1, b1, w_mu, w_log_sigma, eps_w, b_mu,
           b_log_sigma, eps_b):
    n, d = x.shape
    h_dim = W1.shape[1]
    c_dim = w_mu.shape[0]
    e = edge_index.shape[1]
    ept = -(-e // (NW * CHUNK)) * CHUNK
    cpt = ept // CHUNK
    epad = NW * ept
    pad = epad - e

    src = jnp.concatenate([edge_index[0], jnp.zeros((pad,), jnp.int32)])
    dst = jnp.concatenate([edge_index[1], jnp.full((pad,), n, jnp.int32)])
    srcp = src.reshape(NW, cpt, CHUNK)
    dstp = dst.reshape(NW, cpt, CHUNK)

    degp = pl.kernel(
        functools.partial(_deg_body, cpt),
        out_type=jax.ShapeDtypeStruct((NC, NPAD), jnp.float32),
        mesh=_mesh(),
        scratch_types=[
            pltpu.VMEM_SHARED((NPAD,), jnp.float32),
            pltpu.VMEM((cpt, CHUNK), jnp.int32),
            pltpu.VMEM((CHUNK,), jnp.float32),
            pltpu.VMEM((CHUNK,), jnp.float32),
            pltpu.VMEM((RPT,), jnp.float32),
        ],
    )(dstp)

    blk = 1024
    grid = -(-n // blk)
    g = pl.pallas_call(
        _mm_body,
        grid=(grid,),
        in_specs=[
            pl.BlockSpec((blk, d), lambda i: (i, 0)),
            pl.BlockSpec((d, h_dim), lambda i: (0, 0)),
            pl.BlockSpec((NC, blk), lambda i: (0, i)),
        ],
        out_specs=pl.BlockSpec((blk, h_dim), lambda i: (i, 0)),
        out_shape=jax.ShapeDtypeStruct((n, h_dim), jnp.float32),
    )(x, W1, degp)

    s = pl.kernel(
        functools.partial(_scat_body, cpt),
        out_type=jax.ShapeDtypeStruct((NC, NPAD, h_dim), jnp.float32),
        mesh=_mesh(),
        scratch_types=[
            pltpu.VMEM_SHARED((NPAD, h_dim), jnp.float32),
            pltpu.VMEM((cpt, CHUNK), jnp.int32),
            pltpu.VMEM((cpt, CHUNK), jnp.int32),
            pltpu.VMEM((CHUNK, h_dim), jnp.float32),
            pltpu.VMEM((40, h_dim), jnp.float32),
            pltpu.SemaphoreType.DMA,
        ],
    )(srcp, dstp, g)

    out = pl.pallas_call(
        _final_body,
        grid=(grid,),
        in_specs=[
            pl.BlockSpec((NC, blk, h_dim), lambda i: (0, i, 0)),
            pl.BlockSpec((NC, blk), lambda i: (0, i)),
            pl.BlockSpec((blk, h_dim), lambda i: (i, 0)),
            pl.BlockSpec((1, h_dim), lambda i: (0, 0)),
            pl.BlockSpec((c_dim, h_dim), lambda i: (0, 0)),
            pl.BlockSpec((c_dim, h_dim), lambda i: (0, 0)),
            pl.BlockSpec((c_dim, h_dim), lambda i: (0, 0)),
            pl.BlockSpec((1, c_dim), lambda i: (0, 0)),
            pl.BlockSpec((1, c_dim), lambda i: (0, 0)),
            pl.BlockSpec((1, c_dim), lambda i: (0, 0)),
        ],
        out_specs=pl.BlockSpec((blk, c_dim), lambda i: (i, 0)),
        out_shape=jax.ShapeDtypeStruct((n, c_dim), jnp.float32),
    )(s, degp, g, b1.reshape(1, h_dim), w_mu, w_log_sigma, eps_w,
      b_mu.reshape(1, c_dim), b_log_sigma.reshape(1, c_dim),
      eps_b.reshape(1, c_dim))
    return out
